# baseline (speedup 1.0000x reference)
import jax
import jax.numpy as jnp
from jax.experimental import pallas as pl
from jax.experimental.pallas import tpu as pltpu

_ROWS = 4096
_COLS = 16384
_N_STRIPS = 8
_STRIP = _ROWS // _N_STRIPS


def _copy_body(in_hbm, out_hbm, sems):
    for i in range(_N_STRIPS):
        pltpu.make_async_copy(
            in_hbm.at[pl.ds(i * _STRIP, _STRIP), :],
            out_hbm.at[pl.ds(i * _STRIP, _STRIP), :],
            sems.at[i],
        ).start()
    for i in range(_N_STRIPS):
        pltpu.make_async_copy(
            in_hbm.at[pl.ds(i * _STRIP, _STRIP), :],
            out_hbm.at[pl.ds(i * _STRIP, _STRIP), :],
            sems.at[i],
        ).wait()


def kernel(x):
    return pl.pallas_call(
        _copy_body,
        in_specs=[pl.BlockSpec(memory_space=pl.ANY)],
        out_specs=pl.BlockSpec(memory_space=pl.ANY),
        out_shape=jax.ShapeDtypeStruct((_ROWS, _COLS), x.dtype),
        scratch_shapes=[pltpu.SemaphoreType.DMA((_N_STRIPS,))],
    )(x)


# speedup vs baseline: 48.3011x; 48.3011x over previous
import jax
import jax.numpy as jnp
from jax.experimental import pallas as pl

_ROWS = 4096
_COLS = 16384
_BLOCK_ROWS = 64


def _copy_block(in_ref, out_ref):
    out_ref[...] = in_ref[...]


def kernel(x):
    grid = (_ROWS // _BLOCK_ROWS,)
    return pl.pallas_call(
        _copy_block,
        grid=grid,
        in_specs=[pl.BlockSpec((_BLOCK_ROWS, _COLS), lambda i: (i, 0))],
        out_specs=pl.BlockSpec((_BLOCK_ROWS, _COLS), lambda i: (i, 0)),
        out_shape=jax.ShapeDtypeStruct((_ROWS, _COLS), x.dtype),
    )(x)
